# baseline (speedup 1.0000x reference)
import jax
import jax.numpy as jnp
from jax import lax
from jax.experimental import pallas as pl
from jax.experimental.pallas import tpu as pltpu
from jax.experimental.pallas import tpu_sc as plsc

N = 10000
E = 320000
D = 128
NUM_RBF = 16
CUTOFF = 4.0
NTYPES = 8

NC = 2
NS = 16
NW = NC * NS
CB = 80

SLICES = 5
ESL = E // SLICES
EPW = ESL // NW
NCH = EPW // CB
ROWS_SL = ESL // CB
NROWS = N // NS

_STEP = CUTOFF / (NUM_RBF - 1)
_GAMMA = 1.0 / (_STEP * _STEP)

_MESH = plsc.VectorSubcoreMesh(core_axis_name="c", subcore_axis_name="s")
_SC_PARAMS = pltpu.CompilerParams(use_tc_tiling_on_sc=False)


def _precompute_body(h_ref, wa_ref, wb_ref, oa_ref, ob_ref):
    hh = h_ref[...]
    oa_ref[...] = jnp.dot(hh, wa_ref[...], preferred_element_type=jnp.float32)
    ob_ref[...] = jnp.dot(hh, wb_ref[...], preferred_element_type=jnp.float32)


def _precompute(h, w1a, w1b):
    bn = 400
    grid = (N // bn,)
    return pl.pallas_call(
        _precompute_body,
        grid=grid,
        in_specs=[
            pl.BlockSpec((bn, D), lambda i: (i, 0)),
            pl.BlockSpec((D, D), lambda i: (0, 0)),
            pl.BlockSpec((D, D), lambda i: (0, 0)),
        ],
        out_specs=[
            pl.BlockSpec((bn, D), lambda i: (i, 0)),
            pl.BlockSpec((bn, D), lambda i: (i, 0)),
        ],
        out_shape=[
            jax.ShapeDtypeStruct((N, D), jnp.float32),
            jax.ShapeDtypeStruct((N, D), jnp.float32),
        ],
    )(h, w1a, w1b)


def _sc_gather_body(ts, td, c16, c16n, src2, dst2,
                    hsum, rel16,
                    sidx, didx, h0, h1, c0, c1,
                    sh0, sh1, sc0, sc1, sw0, sw1):
    c = lax.axis_index("c")
    s = lax.axis_index("s")
    wid = c * NS + s

    row0 = wid * NCH
    pltpu.sync_copy(src2.at[pl.ds(row0, NCH)], sidx)
    pltpu.sync_copy(dst2.at[pl.ds(row0, NCH)], didx)

    def issue_g(hbuf, cbuf, semh, semc, j):
        si = sidx.at[j]
        pltpu.async_copy(ts.at[si], hbuf, semh)
        pltpu.async_copy(c16.at[si], cbuf, semc)

    def issue_add(hbuf, cbuf, semh, semc, j):
        di = didx.at[j]
        pltpu.async_copy(td.at[di], hbuf, semh, add=True)
        pltpu.async_copy(c16n.at[di], cbuf, semc, add=True)

    def wait_gc(hbuf, cbuf, semh, semc):
        pltpu.make_async_copy(ts.at[pl.ds(0, CB)], hbuf, semh).wait()
        pltpu.make_async_copy(c16.at[pl.ds(0, CB)], cbuf, semc).wait()

    def issue_w(hbuf, cbuf, semw, j):
        e0 = wid * EPW + j * CB
        pltpu.async_copy(hbuf, hsum.at[pl.ds(e0, CB)], semw)
        pltpu.async_copy(cbuf, rel16.at[pl.ds(e0, CB)], semw)

    def wait_w(hbuf, cbuf, semw):
        pltpu.make_async_copy(hbuf, hsum.at[pl.ds(0, CB)], semw).wait()
        pltpu.make_async_copy(cbuf, rel16.at[pl.ds(0, CB)], semw).wait()

    issue_g(h0, c0, sh0, sc0, 0)
    issue_g(h1, c1, sh1, sc1, 1)

    def pair(i, carry):
        j0 = 2 * i
        j1 = 2 * i + 1
        wait_gc(h0, c0, sh0, sc0)
        issue_add(h0, c0, sh0, sc0, j0)
        wait_gc(h0, c0, sh0, sc0)
        issue_w(h0, c0, sw0, j0)
        wait_gc(h1, c1, sh1, sc1)
        issue_add(h1, c1, sh1, sc1, j1)
        wait_w(h0, c0, sw0)
        issue_g(h0, c0, sh0, sc0, j0 + 2)
        wait_gc(h1, c1, sh1, sc1)
        issue_w(h1, c1, sw1, j1)
        wait_w(h1, c1, sw1)
        issue_g(h1, c1, sh1, sc1, jnp.minimum(j1 + 2, NCH - 1))
        return carry

    lax.fori_loop(0, (NCH - 1) // 2, pair, 0)

    jl = NCH - 1
    wait_gc(h0, c0, sh0, sc0)
    issue_add(h0, c0, sh0, sc0, jl)
    wait_gc(h0, c0, sh0, sc0)
    el = wid * EPW + jl * CB
    pltpu.sync_copy(h0, hsum.at[pl.ds(el, CB)])
    pltpu.sync_copy(c0, rel16.at[pl.ds(el, CB)])
    wait_gc(h1, c1, sh1, sc1)


def _sc_gather(ts, td, c16, c16n, src2_s, dst2_s):
    out_type = [
        jax.ShapeDtypeStruct((ESL, D), jnp.float32),
        jax.ShapeDtypeStruct((ESL, 16), jnp.float32),
    ]
    scratch = [
        pltpu.VMEM((NCH, CB), jnp.int32),
        pltpu.VMEM((NCH, CB), jnp.int32),
        pltpu.VMEM((CB, D), jnp.float32),
        pltpu.VMEM((CB, D), jnp.float32),
        pltpu.VMEM((CB, 16), jnp.float32),
        pltpu.VMEM((CB, 16), jnp.float32),
        pltpu.SemaphoreType.DMA,
        pltpu.SemaphoreType.DMA,
        pltpu.SemaphoreType.DMA,
        pltpu.SemaphoreType.DMA,
        pltpu.SemaphoreType.DMA,
        pltpu.SemaphoreType.DMA,
    ]
    return pl.kernel(
        _sc_gather_body,
        out_type=out_type,
        mesh=_MESH,
        scratch_types=scratch,
        compiler_params=_SC_PARAMS,
    )(ts, td, c16, c16n, src2_s, dst2_s)


def _edge_body(hs_ref, rel_ref, et_ref,
               emb_ref, w1e_ref, b1_ref, w1r_ref, w1dt_ref, w2_ref, b2_ref,
               out_ref):
    rel = rel_ref[...]
    r2 = rel * rel
    ones16 = jnp.ones((16, NUM_RBF), jnp.float32)
    s = jnp.dot(r2, ones16, preferred_element_type=jnp.float32)
    dist16 = jnp.sqrt(s)
    centers = lax.broadcasted_iota(
        jnp.int32, (1, NUM_RBF), 1).astype(jnp.float32) * _STEP
    rad = jnp.exp(-_GAMMA * jnp.square(dist16 - centers))
    embw = jnp.dot(emb_ref[...], w1e_ref[...],
                   preferred_element_type=jnp.float32) + b1_ref[...]
    types = lax.broadcasted_iota(jnp.int32, (1, NTYPES), 1)
    oh = (et_ref[...] == types).astype(jnp.float32)
    z = (hs_ref[...]
         + jnp.dot(rad, w1r_ref[...], preferred_element_type=jnp.float32)
         + jnp.dot(dist16, w1dt_ref[...], preferred_element_type=jnp.float32)
         + jnp.dot(oh, embw, preferred_element_type=jnp.float32))
    m1 = z * jax.nn.sigmoid(z)
    m2 = jnp.dot(m1, w2_ref[...], preferred_element_type=jnp.float32) + b2_ref[...]
    out_ref[...] = m2 * jax.nn.sigmoid(m2)


def _edge_mlp(hsum, rel16, et2_s, emb, w1e, b1r, w1r, w1dt, w2, b2r):
    be = 1280
    grid = (ESL // be,)
    return pl.pallas_call(
        _edge_body,
        grid=grid,
        in_specs=[
            pl.BlockSpec((be, D), lambda i: (i, 0)),
            pl.BlockSpec((be, 16), lambda i: (i, 0)),
            pl.BlockSpec((be, 1), lambda i: (i, 0)),
            pl.BlockSpec((NTYPES, D), lambda i: (0, 0)),
            pl.BlockSpec((D, D), lambda i: (0, 0)),
            pl.BlockSpec((1, D), lambda i: (0, 0)),
            pl.BlockSpec((NUM_RBF, D), lambda i: (0, 0)),
            pl.BlockSpec((NUM_RBF, D), lambda i: (0, 0)),
            pl.BlockSpec((D, D), lambda i: (0, 0)),
            pl.BlockSpec((1, D), lambda i: (0, 0)),
        ],
        out_specs=pl.BlockSpec((be, D), lambda i: (i, 0)),
        out_shape=jax.ShapeDtypeStruct((ESL, D), jnp.float32),
    )(hsum, rel16, et2_s, emb, w1e, b1r, w1r, w1dt, w2, b2r)


def _sc_scatter_body(m, dst2, zeros128, zeros_c, ones_c,
                     sums_out, cnt_out,
                     didx, mb0, mb1, ones_v, sums_sh, cnt_sh, sm0, sm1):
    c = lax.axis_index("c")
    s = lax.axis_index("s")
    wid = c * NS + s

    pltpu.sync_copy(zeros128.at[pl.ds(s * NROWS, NROWS)],
                    sums_sh.at[pl.ds(s * NROWS, NROWS)])

    @pl.when(s == 0)
    def _():
        pltpu.sync_copy(zeros_c, cnt_sh)

    pltpu.sync_copy(ones_c, ones_v)
    row0 = wid * NCH
    pltpu.sync_copy(dst2.at[pl.ds(row0, NCH)], didx)
    plsc.subcore_barrier()

    def issue_l(mbuf, sem, j):
        e0 = wid * EPW + j * CB
        pltpu.async_copy(m.at[pl.ds(e0, CB)], mbuf, sem)

    def wait_l(mbuf, sem):
        pltpu.make_async_copy(m.at[pl.ds(0, CB)], mbuf, sem).wait()

    def scat(mbuf, j):
        di = didx.at[j]
        pltpu.sync_copy(mbuf, sums_sh.at[di], add=True)
        pltpu.sync_copy(ones_v, cnt_sh.at[di], add=True)

    issue_l(mb0, sm0, 0)

    def pair(i, carry):
        j0 = 2 * i
        j1 = 2 * i + 1
        issue_l(mb1, sm1, j1)
        wait_l(mb0, sm0)
        scat(mb0, j0)
        issue_l(mb0, sm0, jnp.minimum(j0 + 2, NCH - 1))
        wait_l(mb1, sm1)
        scat(mb1, j1)
        return carry

    lax.fori_loop(0, (NCH - 1) // 2, pair, 0)
    wait_l(mb0, sm0)
    scat(mb0, NCH - 1)
    plsc.subcore_barrier()
    pltpu.sync_copy(sums_sh.at[pl.ds(s * NROWS, NROWS)],
                    sums_out.at[c, pl.ds(s * NROWS, NROWS)])

    @pl.when(s == 0)
    def _():
        pltpu.sync_copy(cnt_sh, cnt_out.at[c])


def _sc_scatter(m_s, dst2_s):
    zeros128 = jnp.zeros((N, D), jnp.float32)
    zeros_c = jnp.zeros((N, 16), jnp.float32)
    ones_c = jnp.ones((CB, 16), jnp.float32)
    out_type = [
        jax.ShapeDtypeStruct((NC, N, D), jnp.float32),
        jax.ShapeDtypeStruct((NC, N, 16), jnp.float32),
    ]
    scratch = [
        pltpu.VMEM((NCH, CB), jnp.int32),
        pltpu.VMEM((CB, D), jnp.float32),
        pltpu.VMEM((CB, D), jnp.float32),
        pltpu.VMEM((CB, 16), jnp.float32),
        pltpu.VMEM_SHARED((N, D), jnp.float32),
        pltpu.VMEM_SHARED((N, 16), jnp.float32),
        pltpu.SemaphoreType.DMA,
        pltpu.SemaphoreType.DMA,
    ]
    return pl.kernel(
        _sc_scatter_body,
        out_type=out_type,
        mesh=_MESH,
        scratch_types=scratch,
        compiler_params=_SC_PARAMS,
    )(m_s, dst2_s, zeros128, zeros_c, ones_c)


def _node_body(h_ref, s0_ref, s1_ref, s2_ref, s3_ref, s4_ref,
               c0_ref, c1_ref, c2_ref, c3_ref, c4_ref,
               u1a_ref, u1b_ref, ub1_ref, u2_ref, ub2_ref, g_ref, b_ref,
               out_ref):
    hh = h_ref[...]
    cnt = (c0_ref[0, :, :1] + c0_ref[1, :, :1]
           + c1_ref[0, :, :1] + c1_ref[1, :, :1]
           + c2_ref[0, :, :1] + c2_ref[1, :, :1]
           + c3_ref[0, :, :1] + c3_ref[1, :, :1]
           + c4_ref[0, :, :1] + c4_ref[1, :, :1])
    cnt = jnp.maximum(cnt, 1.0)
    tot = (s0_ref[0] + s0_ref[1] + s1_ref[0] + s1_ref[1]
           + s2_ref[0] + s2_ref[1] + s3_ref[0] + s3_ref[1]
           + s4_ref[0] + s4_ref[1])
    agg = tot / cnt
    t = (jnp.dot(hh, u1a_ref[...], preferred_element_type=jnp.float32)
         + jnp.dot(agg, u1b_ref[...], preferred_element_type=jnp.float32)
         + ub1_ref[...])
    t = t * jax.nn.sigmoid(t)
    u = jnp.dot(t, u2_ref[...], preferred_element_type=jnp.float32) + ub2_ref[...]
    x = hh + u
    mu = jnp.mean(x, axis=-1, keepdims=True)
    var = jnp.mean(jnp.square(x - mu), axis=-1, keepdims=True)
    out_ref[...] = (x - mu) / jnp.sqrt(var + 1e-5) * g_ref[...] + b_ref[...]


def _node_update(h, sums_list, cnt_list, u1a, u1b, ub1r, u2, ub2r, gr, br):
    bn = 400
    grid = (N // bn,)
    sum_spec = pl.BlockSpec((NC, bn, D), lambda i: (0, i, 0))
    cnt_spec = pl.BlockSpec((NC, bn, 16), lambda i: (0, i, 0))
    full = lambda shape: pl.BlockSpec(shape, lambda i: (0, 0))
    return pl.pallas_call(
        _node_body,
        grid=grid,
        in_specs=[pl.BlockSpec((bn, D), lambda i: (i, 0))]
        + [sum_spec] * SLICES + [cnt_spec] * SLICES
        + [full((D, D)), full((D, D)), full((1, D)),
           full((D, D)), full((1, D)), full((1, D)), full((1, D))],
        out_specs=pl.BlockSpec((bn, D), lambda i: (i, 0)),
        out_shape=jax.ShapeDtypeStruct((N, D), jnp.float32),
    )(h, *sums_list, *cnt_list, u1a, u1b, ub1r, u2, ub2r, gr, br)


def kernel(h, coords, edge_index, edge_type, emb, W1, b1, W2, b2,
           U1, ub1, U2, ub2, ln_g, ln_b):
    src = edge_index[0].astype(jnp.int32)
    dst = edge_index[1].astype(jnp.int32)
    src2 = src.reshape(E // CB, CB)
    dst2 = dst.reshape(E // CB, CB)
    et2 = edge_type.astype(jnp.int32).reshape(E, 1)

    w1a = W1[:D]
    w1b = W1[D:2 * D]
    w1e = W1[2 * D:3 * D]
    w1r = W1[3 * D:3 * D + NUM_RBF]
    w1dr = W1[3 * D + NUM_RBF:]
    w1dt = jnp.tile(w1dr / 16.0, (16, 1))
    b1r = b1.reshape(1, D)
    b2r = b2.reshape(1, D)
    u1a = U1[:D]
    u1b = U1[D:]
    ub1r = ub1.reshape(1, D)
    ub2r = ub2.reshape(1, D)
    gr = ln_g.reshape(1, D)
    br = ln_b.reshape(1, D)

    hws, hwd = _precompute(h, w1a, w1b)
    cpad = jnp.zeros((N, 13), jnp.float32)
    c16 = jnp.concatenate([coords.astype(jnp.float32), cpad], axis=1)
    c16n = -c16

    sums_list = []
    cnt_list = []
    for sl in range(SLICES):
        src2_s = src2[sl * ROWS_SL:(sl + 1) * ROWS_SL]
        dst2_s = dst2[sl * ROWS_SL:(sl + 1) * ROWS_SL]
        et2_s = et2[sl * ESL:(sl + 1) * ESL]
        hsum, rel16 = _sc_gather(hws, hwd, c16, c16n, src2_s, dst2_s)
        m_s = _edge_mlp(hsum, rel16, et2_s, emb, w1e, b1r, w1r, w1dt, W2, b2r)
        s_s, c_s = _sc_scatter(m_s, dst2_s)
        sums_list.append(s_s)
        cnt_list.append(c_s)

    return _node_update(h, sums_list, cnt_list, u1a, u1b, ub1r, U2, ub2r, gr, br)


# speedup vs baseline: 1.0283x; 1.0283x over previous
import jax
import jax.numpy as jnp
from jax import lax
from jax.experimental import pallas as pl
from jax.experimental.pallas import tpu as pltpu
from jax.experimental.pallas import tpu_sc as plsc

N = 10000
E = 320000
D = 128
NUM_RBF = 16
CUTOFF = 4.0
NTYPES = 8

NC = 2
NS = 16
NW = NC * NS
CB = 80

SLICES = 5
ESL = E // SLICES
EPW = ESL // NW
NCH = EPW // CB
ROWS_SL = ESL // CB
NROWS = N // NS

_STEP = CUTOFF / (NUM_RBF - 1)
_GAMMA = 1.0 / (_STEP * _STEP)

_MESH = plsc.VectorSubcoreMesh(core_axis_name="c", subcore_axis_name="s")
_SC_PARAMS = pltpu.CompilerParams(use_tc_tiling_on_sc=False)


def _precompute_body(h_ref, wa_ref, wb_ref, oa_ref, ob_ref):
    hh = h_ref[...]
    oa_ref[...] = jnp.dot(hh, wa_ref[...], preferred_element_type=jnp.float32)
    ob_ref[...] = jnp.dot(hh, wb_ref[...], preferred_element_type=jnp.float32)


def _precompute(h, w1a, w1b):
    bn = 400
    grid = (N // bn,)
    return pl.pallas_call(
        _precompute_body,
        grid=grid,
        in_specs=[
            pl.BlockSpec((bn, D), lambda i: (i, 0)),
            pl.BlockSpec((D, D), lambda i: (0, 0)),
            pl.BlockSpec((D, D), lambda i: (0, 0)),
        ],
        out_specs=[
            pl.BlockSpec((bn, D), lambda i: (i, 0)),
            pl.BlockSpec((bn, D), lambda i: (i, 0)),
        ],
        out_shape=[
            jax.ShapeDtypeStruct((N, D), jnp.float32),
            jax.ShapeDtypeStruct((N, D), jnp.float32),
        ],
    )(h, w1a, w1b)


def _sc_gather_body(ts, td, c16, c16n, src2, dst2,
                    hsum, rel16,
                    sidx, didx, h0, h1, c0, c1, ts_sh, c16_sh,
                    sh0, sh1, sc0, sc1, sw0, sw1):
    c = lax.axis_index("c")
    s = lax.axis_index("s")
    wid = c * NS + s

    pltpu.sync_copy(ts.at[pl.ds(s * NROWS, NROWS)],
                    ts_sh.at[pl.ds(s * NROWS, NROWS)])
    pltpu.sync_copy(c16.at[pl.ds(s * NROWS, NROWS)],
                    c16_sh.at[pl.ds(s * NROWS, NROWS)])
    row0 = wid * NCH
    pltpu.sync_copy(src2.at[pl.ds(row0, NCH)], sidx)
    pltpu.sync_copy(dst2.at[pl.ds(row0, NCH)], didx)
    plsc.subcore_barrier()

    def issue_g(hbuf, cbuf, semh, semc, j):
        si = sidx.at[j]
        pltpu.async_copy(ts_sh.at[si], hbuf, semh)
        pltpu.async_copy(c16_sh.at[si], cbuf, semc)

    def issue_add(hbuf, cbuf, semh, semc, j):
        di = didx.at[j]
        pltpu.async_copy(td.at[di], hbuf, semh, add=True)
        pltpu.async_copy(c16n.at[di], cbuf, semc, add=True)

    def wait_gc(hbuf, cbuf, semh, semc):
        pltpu.make_async_copy(ts.at[pl.ds(0, CB)], hbuf, semh).wait()
        pltpu.make_async_copy(c16.at[pl.ds(0, CB)], cbuf, semc).wait()


    def issue_w(hbuf, cbuf, semw, j):
        e0 = wid * EPW + j * CB
        pltpu.async_copy(hbuf, hsum.at[pl.ds(e0, CB)], semw)
        pltpu.async_copy(cbuf, rel16.at[pl.ds(e0, CB)], semw)

    def wait_w(hbuf, cbuf, semw):
        pltpu.make_async_copy(hbuf, hsum.at[pl.ds(0, CB)], semw).wait()
        pltpu.make_async_copy(cbuf, rel16.at[pl.ds(0, CB)], semw).wait()

    issue_g(h0, c0, sh0, sc0, 0)
    issue_g(h1, c1, sh1, sc1, 1)

    def pair(i, carry):
        j0 = 2 * i
        j1 = 2 * i + 1
        wait_gc(h0, c0, sh0, sc0)
        issue_add(h0, c0, sh0, sc0, j0)
        wait_gc(h0, c0, sh0, sc0)
        issue_w(h0, c0, sw0, j0)
        wait_gc(h1, c1, sh1, sc1)
        issue_add(h1, c1, sh1, sc1, j1)
        wait_w(h0, c0, sw0)
        issue_g(h0, c0, sh0, sc0, j0 + 2)
        wait_gc(h1, c1, sh1, sc1)
        issue_w(h1, c1, sw1, j1)
        wait_w(h1, c1, sw1)
        issue_g(h1, c1, sh1, sc1, jnp.minimum(j1 + 2, NCH - 1))
        return carry

    lax.fori_loop(0, (NCH - 1) // 2, pair, 0)

    jl = NCH - 1
    wait_gc(h0, c0, sh0, sc0)
    issue_add(h0, c0, sh0, sc0, jl)
    wait_gc(h0, c0, sh0, sc0)
    el = wid * EPW + jl * CB
    pltpu.sync_copy(h0, hsum.at[pl.ds(el, CB)])
    pltpu.sync_copy(c0, rel16.at[pl.ds(el, CB)])
    wait_gc(h1, c1, sh1, sc1)


def _sc_gather(ts, td, c16, c16n, src2_s, dst2_s):
    out_type = [
        jax.ShapeDtypeStruct((ESL, D), jnp.float32),
        jax.ShapeDtypeStruct((ESL, 16), jnp.float32),
    ]
    scratch = [
        pltpu.VMEM((NCH, CB), jnp.int32),
        pltpu.VMEM((NCH, CB), jnp.int32),
        pltpu.VMEM((CB, D), jnp.float32),
        pltpu.VMEM((CB, D), jnp.float32),
        pltpu.VMEM((CB, 16), jnp.float32),
        pltpu.VMEM((CB, 16), jnp.float32),
        pltpu.VMEM_SHARED((N, D), jnp.float32),
        pltpu.VMEM_SHARED((N, 16), jnp.float32),
        pltpu.SemaphoreType.DMA,
        pltpu.SemaphoreType.DMA,
        pltpu.SemaphoreType.DMA,
        pltpu.SemaphoreType.DMA,
        pltpu.SemaphoreType.DMA,
        pltpu.SemaphoreType.DMA,
    ]
    return pl.kernel(
        _sc_gather_body,
        out_type=out_type,
        mesh=_MESH,
        scratch_types=scratch,
        compiler_params=_SC_PARAMS,
    )(ts, td, c16, c16n, src2_s, dst2_s)


def _edge_body(hs_ref, rel_ref, et_ref,
               emb_ref, w1e_ref, b1_ref, w1r_ref, w1dt_ref, w2_ref, b2_ref,
               out_ref):
    rel = rel_ref[...]
    r2 = rel * rel
    ones16 = jnp.ones((16, NUM_RBF), jnp.float32)
    s = jnp.dot(r2, ones16, preferred_element_type=jnp.float32)
    dist16 = jnp.sqrt(s)
    centers = lax.broadcasted_iota(
        jnp.int32, (1, NUM_RBF), 1).astype(jnp.float32) * _STEP
    rad = jnp.exp(-_GAMMA * jnp.square(dist16 - centers))
    embw = jnp.dot(emb_ref[...], w1e_ref[...],
                   preferred_element_type=jnp.float32) + b1_ref[...]
    types = lax.broadcasted_iota(jnp.int32, (1, NTYPES), 1)
    oh = (et_ref[...] == types).astype(jnp.float32)
    z = (hs_ref[...]
         + jnp.dot(rad, w1r_ref[...], preferred_element_type=jnp.float32)
         + jnp.dot(dist16, w1dt_ref[...], preferred_element_type=jnp.float32)
         + jnp.dot(oh, embw, preferred_element_type=jnp.float32))
    m1 = z * jax.nn.sigmoid(z)
    m2 = jnp.dot(m1, w2_ref[...], preferred_element_type=jnp.float32) + b2_ref[...]
    out_ref[...] = m2 * jax.nn.sigmoid(m2)


def _edge_mlp(hsum, rel16, et2_s, emb, w1e, b1r, w1r, w1dt, w2, b2r):
    be = 1280
    grid = (ESL // be,)
    return pl.pallas_call(
        _edge_body,
        grid=grid,
        in_specs=[
            pl.BlockSpec((be, D), lambda i: (i, 0)),
            pl.BlockSpec((be, 16), lambda i: (i, 0)),
            pl.BlockSpec((be, 1), lambda i: (i, 0)),
            pl.BlockSpec((NTYPES, D), lambda i: (0, 0)),
            pl.BlockSpec((D, D), lambda i: (0, 0)),
            pl.BlockSpec((1, D), lambda i: (0, 0)),
            pl.BlockSpec((NUM_RBF, D), lambda i: (0, 0)),
            pl.BlockSpec((NUM_RBF, D), lambda i: (0, 0)),
            pl.BlockSpec((D, D), lambda i: (0, 0)),
            pl.BlockSpec((1, D), lambda i: (0, 0)),
        ],
        out_specs=pl.BlockSpec((be, D), lambda i: (i, 0)),
        out_shape=jax.ShapeDtypeStruct((ESL, D), jnp.float32),
    )(hsum, rel16, et2_s, emb, w1e, b1r, w1r, w1dt, w2, b2r)


def _sc_scatter_body(m, dst2, zeros128, zeros_c, ones_c,
                     sums_out, cnt_out,
                     didx, mb0, mb1, ones_v, sums_sh, cnt_sh, sm0, sm1):
    c = lax.axis_index("c")
    s = lax.axis_index("s")
    wid = c * NS + s

    pltpu.sync_copy(zeros128.at[pl.ds(s * NROWS, NROWS)],
                    sums_sh.at[pl.ds(s * NROWS, NROWS)])

    @pl.when(s == 0)
    def _():
        pltpu.sync_copy(zeros_c, cnt_sh)

    pltpu.sync_copy(ones_c, ones_v)
    row0 = wid * NCH
    pltpu.sync_copy(dst2.at[pl.ds(row0, NCH)], didx)
    plsc.subcore_barrier()

    def issue_l(mbuf, sem, j):
        e0 = wid * EPW + j * CB
        pltpu.async_copy(m.at[pl.ds(e0, CB)], mbuf, sem)

    def wait_l(mbuf, sem):
        pltpu.make_async_copy(m.at[pl.ds(0, CB)], mbuf, sem).wait()

    def scat(mbuf, j):
        di = didx.at[j]
        pltpu.sync_copy(mbuf, sums_sh.at[di], add=True)
        pltpu.sync_copy(ones_v, cnt_sh.at[di], add=True)

    issue_l(mb0, sm0, 0)

    def pair(i, carry):
        j0 = 2 * i
        j1 = 2 * i + 1
        issue_l(mb1, sm1, j1)
        wait_l(mb0, sm0)
        scat(mb0, j0)
        issue_l(mb0, sm0, jnp.minimum(j0 + 2, NCH - 1))
        wait_l(mb1, sm1)
        scat(mb1, j1)
        return carry

    lax.fori_loop(0, (NCH - 1) // 2, pair, 0)
    wait_l(mb0, sm0)
    scat(mb0, NCH - 1)
    plsc.subcore_barrier()
    pltpu.sync_copy(sums_sh.at[pl.ds(s * NROWS, NROWS)],
                    sums_out.at[c, pl.ds(s * NROWS, NROWS)])

    @pl.when(s == 0)
    def _():
        pltpu.sync_copy(cnt_sh, cnt_out.at[c])


def _sc_scatter(m_s, dst2_s):
    zeros128 = jnp.zeros((N, D), jnp.float32)
    zeros_c = jnp.zeros((N, 16), jnp.float32)
    ones_c = jnp.ones((CB, 16), jnp.float32)
    out_type = [
        jax.ShapeDtypeStruct((NC, N, D), jnp.float32),
        jax.ShapeDtypeStruct((NC, N, 16), jnp.float32),
    ]
    scratch = [
        pltpu.VMEM((NCH, CB), jnp.int32),
        pltpu.VMEM((CB, D), jnp.float32),
        pltpu.VMEM((CB, D), jnp.float32),
        pltpu.VMEM((CB, 16), jnp.float32),
        pltpu.VMEM_SHARED((N, D), jnp.float32),
        pltpu.VMEM_SHARED((N, 16), jnp.float32),
        pltpu.SemaphoreType.DMA,
        pltpu.SemaphoreType.DMA,
    ]
    return pl.kernel(
        _sc_scatter_body,
        out_type=out_type,
        mesh=_MESH,
        scratch_types=scratch,
        compiler_params=_SC_PARAMS,
    )(m_s, dst2_s, zeros128, zeros_c, ones_c)


def _node_body(h_ref, s0_ref, s1_ref, s2_ref, s3_ref, s4_ref,
               c0_ref, c1_ref, c2_ref, c3_ref, c4_ref,
               u1a_ref, u1b_ref, ub1_ref, u2_ref, ub2_ref, g_ref, b_ref,
               out_ref):
    hh = h_ref[...]
    cnt = (c0_ref[0, :, :1] + c0_ref[1, :, :1]
           + c1_ref[0, :, :1] + c1_ref[1, :, :1]
           + c2_ref[0, :, :1] + c2_ref[1, :, :1]
           + c3_ref[0, :, :1] + c3_ref[1, :, :1]
           + c4_ref[0, :, :1] + c4_ref[1, :, :1])
    cnt = jnp.maximum(cnt, 1.0)
    tot = (s0_ref[0] + s0_ref[1] + s1_ref[0] + s1_ref[1]
           + s2_ref[0] + s2_ref[1] + s3_ref[0] + s3_ref[1]
           + s4_ref[0] + s4_ref[1])
    agg = tot / cnt
    t = (jnp.dot(hh, u1a_ref[...], preferred_element_type=jnp.float32)
         + jnp.dot(agg, u1b_ref[...], preferred_element_type=jnp.float32)
         + ub1_ref[...])
    t = t * jax.nn.sigmoid(t)
    u = jnp.dot(t, u2_ref[...], preferred_element_type=jnp.float32) + ub2_ref[...]
    x = hh + u
    mu = jnp.mean(x, axis=-1, keepdims=True)
    var = jnp.mean(jnp.square(x - mu), axis=-1, keepdims=True)
    out_ref[...] = (x - mu) / jnp.sqrt(var + 1e-5) * g_ref[...] + b_ref[...]


def _node_update(h, sums_list, cnt_list, u1a, u1b, ub1r, u2, ub2r, gr, br):
    bn = 400
    grid = (N // bn,)
    sum_spec = pl.BlockSpec((NC, bn, D), lambda i: (0, i, 0))
    cnt_spec = pl.BlockSpec((NC, bn, 16), lambda i: (0, i, 0))
    full = lambda shape: pl.BlockSpec(shape, lambda i: (0, 0))
    return pl.pallas_call(
        _node_body,
        grid=grid,
        in_specs=[pl.BlockSpec((bn, D), lambda i: (i, 0))]
        + [sum_spec] * SLICES + [cnt_spec] * SLICES
        + [full((D, D)), full((D, D)), full((1, D)),
           full((D, D)), full((1, D)), full((1, D)), full((1, D))],
        out_specs=pl.BlockSpec((bn, D), lambda i: (i, 0)),
        out_shape=jax.ShapeDtypeStruct((N, D), jnp.float32),
    )(h, *sums_list, *cnt_list, u1a, u1b, ub1r, u2, ub2r, gr, br)


def kernel(h, coords, edge_index, edge_type, emb, W1, b1, W2, b2,
           U1, ub1, U2, ub2, ln_g, ln_b):
    src = edge_index[0].astype(jnp.int32)
    dst = edge_index[1].astype(jnp.int32)
    src2 = src.reshape(E // CB, CB)
    dst2 = dst.reshape(E // CB, CB)
    et2 = edge_type.astype(jnp.int32).reshape(E, 1)

    w1a = W1[:D]
    w1b = W1[D:2 * D]
    w1e = W1[2 * D:3 * D]
    w1r = W1[3 * D:3 * D + NUM_RBF]
    w1dr = W1[3 * D + NUM_RBF:]
    w1dt = jnp.tile(w1dr / 16.0, (16, 1))
    b1r = b1.reshape(1, D)
    b2r = b2.reshape(1, D)
    u1a = U1[:D]
    u1b = U1[D:]
    ub1r = ub1.reshape(1, D)
    ub2r = ub2.reshape(1, D)
    gr = ln_g.reshape(1, D)
    br = ln_b.reshape(1, D)

    hws, hwd = _precompute(h, w1a, w1b)
    cpad = jnp.zeros((N, 13), jnp.float32)
    c16 = jnp.concatenate([coords.astype(jnp.float32), cpad], axis=1)
    c16n = -c16

    sums_list = []
    cnt_list = []
    for sl in range(SLICES):
        src2_s = src2[sl * ROWS_SL:(sl + 1) * ROWS_SL]
        dst2_s = dst2[sl * ROWS_SL:(sl + 1) * ROWS_SL]
        et2_s = et2[sl * ESL:(sl + 1) * ESL]
        hsum, rel16 = _sc_gather(hws, hwd, c16, c16n, src2_s, dst2_s)
        m_s = _edge_mlp(hsum, rel16, et2_s, emb, w1e, b1r, w1r, w1dt, W2, b2r)
        s_s, c_s = _sc_scatter(m_s, dst2_s)
        sums_list.append(s_s)
        cnt_list.append(c_s)

    return _node_update(h, sums_list, cnt_list, u1a, u1b, ub1r, U2, ub2r, gr, br)
